# baseline (speedup 1.0000x reference)
import functools

import jax
import jax.numpy as jnp
from jax import lax
from jax.experimental import pallas as pl
from jax.experimental.pallas import tpu as pltpu
from jax.experimental.pallas import tpu_sc as plsc

VOCAB = 1000000
EMBED_DIM = 64
OUT_DIM = 2
SEQ_LEN = 200
BATCH = 4096

_LB = 32768
_NSLOT = 10


def _mm_body(w_ref, et_ref, o_ref):
    r = lax.dot_general(w_ref[...], et_ref[...],
                        (((0,), (0,)), ((), ())),
                        preferred_element_type=jnp.float32)
    r = r * (1.0 / SEQ_LEN)
    u = lax.bitcast_convert_type(r.astype(jnp.bfloat16), jnp.uint16)
    lo = u[0].astype(jnp.uint32)
    hi = u[1].astype(jnp.uint32)
    o_ref[...] = lax.bitcast_convert_type((hi << 16) | lo, jnp.int32)


def _project(et, w):
    return pl.pallas_call(
        _mm_body,
        grid=(pl.cdiv(VOCAB, _LB),),
        in_specs=[
            pl.BlockSpec((EMBED_DIM, OUT_DIM), lambda i: (0, 0)),
            pl.BlockSpec((EMBED_DIM, _LB), lambda i: (0, i)),
        ],
        out_specs=pl.BlockSpec((_LB,), lambda i: (i,)),
        out_shape=jax.ShapeDtypeStruct((VOCAB,), jnp.int32),
    )(w, et)


def _make_sc_pool():
    info = plsc.get_sparse_core_info()
    nc, ns = info.num_cores, info.num_subcores
    nw = nc * ns
    bpw = BATCH // nw
    mesh = plsc.VectorSubcoreMesh(core_axis_name="c", subcore_axis_name="s")

    @functools.partial(
        pl.kernel,
        out_type=jax.ShapeDtypeStruct((OUT_DIM, BATCH), jnp.float32),
        mesh=mesh,
        compiler_params=pltpu.CompilerParams(use_tc_tiling_on_sc=False,
                                             needs_layout_passes=False),
        scratch_types=[
            pltpu.VMEM((SEQ_LEN, bpw), jnp.int32),
            *[pltpu.VMEM((bpw,), jnp.int32) for _ in range(_NSLOT)],
            pltpu.VMEM((bpw,), jnp.float32),
            pltpu.VMEM((bpw,), jnp.float32),
            *[pltpu.SemaphoreType.DMA for _ in range(_NSLOT)],
        ],
    )
    def sc_pool(text_hbm, pk_hbm, out_hbm, idx_v, *rest):
        bufs = rest[:_NSLOT]
        acc0 = rest[_NSLOT]
        acc1 = rest[_NSLOT + 1]
        sems = rest[_NSLOT + 2:]
        wid = lax.axis_index("s") * nc + lax.axis_index("c")
        base = wid * bpw

        pltpu.sync_copy(text_hbm.at[:, pl.ds(base, bpw)], idx_v)

        zero = jnp.zeros((16,), jnp.float32)
        for u in range(bpw // 16):
            acc0[pl.ds(u * 16, 16)] = zero
            acc1[pl.ds(u * 16, 16)] = zero

        hi_mask = jnp.full((16,), -65536, jnp.int32)

        def issue(s, j):
            pltpu.async_copy(pk_hbm.at[idx_v.at[s]], bufs[j], sems[j])

        for j in range(_NSLOT):
            issue(j, j)

        def body(g, carry):
            for j in range(_NSLOT):
                v = g * _NSLOT + j
                pltpu.make_async_copy(
                    pk_hbm.at[idx_v.at[v]], bufs[j], sems[j]).wait()
                for u in range(bpw // 16):
                    sl = pl.ds(u * 16, 16)
                    w = bufs[j][sl]
                    p0 = plsc.bitcast(w << 16, jnp.float32)
                    p1 = plsc.bitcast(w & hi_mask, jnp.float32)
                    plsc.addupdate(acc0.at[sl], p0)
                    plsc.addupdate(acc1.at[sl], p1)

                @pl.when(v + _NSLOT < SEQ_LEN)
                def _():
                    issue(v + _NSLOT, j)

            return carry

        lax.fori_loop(0, SEQ_LEN // _NSLOT, body, 0)

        pltpu.sync_copy(acc0, out_hbm.at[0, pl.ds(base, bpw)])
        pltpu.sync_copy(acc1, out_hbm.at[1, pl.ds(base, bpw)])

    return sc_pool


_sc_pool = None


def kernel(text, embedding, W, b):
    global _sc_pool
    if _sc_pool is None:
        _sc_pool = _make_sc_pool()
    pk = _project(embedding.T, W)
    out2 = _sc_pool(text, pk)
    return out2.T + b


# speedup vs baseline: 1.0066x; 1.0066x over previous
import functools

import jax
import jax.numpy as jnp
from jax import lax
from jax.experimental import pallas as pl
from jax.experimental.pallas import tpu as pltpu
from jax.experimental.pallas import tpu_sc as plsc

VOCAB = 1000000
EMBED_DIM = 64
OUT_DIM = 2
SEQ_LEN = 200
BATCH = 4096

_LB = 32768
_NSLOT = 8


def _mm_body(w_ref, et_ref, o_ref):
    r = lax.dot_general(w_ref[...], et_ref[...],
                        (((0,), (0,)), ((), ())),
                        preferred_element_type=jnp.float32)
    r = r * (1.0 / SEQ_LEN)
    u = lax.bitcast_convert_type(r.astype(jnp.bfloat16), jnp.uint16)
    lo = u[0].astype(jnp.uint32)
    hi = u[1].astype(jnp.uint32)
    o_ref[...] = lax.bitcast_convert_type((hi << 16) | lo, jnp.int32)


def _project(et, w):
    return pl.pallas_call(
        _mm_body,
        grid=(pl.cdiv(VOCAB, _LB),),
        in_specs=[
            pl.BlockSpec((EMBED_DIM, OUT_DIM), lambda i: (0, 0)),
            pl.BlockSpec((EMBED_DIM, _LB), lambda i: (0, i)),
        ],
        out_specs=pl.BlockSpec((_LB,), lambda i: (i,)),
        out_shape=jax.ShapeDtypeStruct((VOCAB,), jnp.int32),
    )(w, et)


def _make_sc_pool():
    info = plsc.get_sparse_core_info()
    nc, ns = info.num_cores, info.num_subcores
    nw = nc * ns
    bpw = BATCH // nw
    mesh = plsc.VectorSubcoreMesh(core_axis_name="c", subcore_axis_name="s")

    @functools.partial(
        pl.kernel,
        out_type=jax.ShapeDtypeStruct((OUT_DIM, BATCH), jnp.float32),
        mesh=mesh,
        compiler_params=pltpu.CompilerParams(use_tc_tiling_on_sc=False,
                                             needs_layout_passes=False),
        scratch_types=[
            pltpu.VMEM((SEQ_LEN, bpw), jnp.int32),
            *[pltpu.VMEM((bpw,), jnp.int32) for _ in range(_NSLOT)],
            pltpu.VMEM((bpw,), jnp.float32),
            pltpu.VMEM((bpw,), jnp.float32),
            *[pltpu.SemaphoreType.DMA for _ in range(_NSLOT)],
        ],
    )
    def sc_pool(text_hbm, pk_hbm, out_hbm, idx_v, *rest):
        bufs = rest[:_NSLOT]
        acc0 = rest[_NSLOT]
        acc1 = rest[_NSLOT + 1]
        sems = rest[_NSLOT + 2:]
        wid = lax.axis_index("s") * nc + lax.axis_index("c")
        base = wid * bpw

        pltpu.sync_copy(text_hbm.at[:, pl.ds(base, bpw)], idx_v)

        zero = jnp.zeros((16,), jnp.float32)
        for u in range(bpw // 16):
            acc0[pl.ds(u * 16, 16)] = zero
            acc1[pl.ds(u * 16, 16)] = zero

        hi_mask = jnp.full((16,), -65536, jnp.int32)

        def issue(s, j):
            pltpu.async_copy(pk_hbm.at[idx_v.at[s]], bufs[j], sems[j])

        for j in range(_NSLOT):
            issue(j, j)

        def body(g, carry):
            for j in range(_NSLOT):
                v = g * _NSLOT + j
                pltpu.make_async_copy(
                    pk_hbm.at[idx_v.at[v]], bufs[j], sems[j]).wait()
                for u in range(bpw // 16):
                    sl = pl.ds(u * 16, 16)
                    w = bufs[j][sl]
                    p0 = plsc.bitcast(w << 16, jnp.float32)
                    p1 = plsc.bitcast(w & hi_mask, jnp.float32)
                    plsc.addupdate(acc0.at[sl], p0)
                    plsc.addupdate(acc1.at[sl], p1)

                @pl.when(v + _NSLOT < SEQ_LEN)
                def _():
                    issue(v + _NSLOT, j)

            return carry

        lax.fori_loop(0, SEQ_LEN // _NSLOT, body, 0)

        pltpu.sync_copy(acc0, out_hbm.at[0, pl.ds(base, bpw)])
        pltpu.sync_copy(acc1, out_hbm.at[1, pl.ds(base, bpw)])

    return sc_pool


_sc_pool = None


def kernel(text, embedding, W, b):
    global _sc_pool
    if _sc_pool is None:
        _sc_pool = _make_sc_pool()
    pk = _project(embedding.T, W)
    out2 = _sc_pool(text, pk)
    return out2.T + b


# speedup vs baseline: 1.0406x; 1.0337x over previous
import functools

import jax
import jax.numpy as jnp
from jax import lax
from jax.experimental import pallas as pl
from jax.experimental.pallas import tpu as pltpu
from jax.experimental.pallas import tpu_sc as plsc

VOCAB = 1000000
EMBED_DIM = 64
OUT_DIM = 2
SEQ_LEN = 200
BATCH = 4096

_LB = 32768
_NSLOT = 8


def _mm_body(w_ref, et_ref, o_ref):
    r = lax.dot_general(w_ref[...], et_ref[...],
                        (((0,), (0,)), ((), ())),
                        preferred_element_type=jnp.float32)
    r = r * (1.0 / SEQ_LEN)
    u = lax.bitcast_convert_type(r.astype(jnp.bfloat16), jnp.uint16)
    lo = u[0].astype(jnp.uint32)
    hi = u[1].astype(jnp.uint32)
    o_ref[...] = lax.bitcast_convert_type((hi << 16) | lo, jnp.int32)


def _project(et, w):
    return pl.pallas_call(
        _mm_body,
        grid=(pl.cdiv(VOCAB, _LB),),
        in_specs=[
            pl.BlockSpec((EMBED_DIM, OUT_DIM), lambda i: (0, 0)),
            pl.BlockSpec((EMBED_DIM, _LB), lambda i: (0, i)),
        ],
        out_specs=pl.BlockSpec((_LB,), lambda i: (i,)),
        out_shape=jax.ShapeDtypeStruct((VOCAB,), jnp.int32),
    )(w, et)


def _make_sc_pool():
    info = plsc.get_sparse_core_info()
    nc, ns = info.num_cores, info.num_subcores
    nw = nc * ns
    bpw = BATCH // nw
    mesh = plsc.VectorSubcoreMesh(core_axis_name="c", subcore_axis_name="s")

    @functools.partial(
        pl.kernel,
        out_type=jax.ShapeDtypeStruct((OUT_DIM, BATCH), jnp.float32),
        mesh=mesh,
        compiler_params=pltpu.CompilerParams(use_tc_tiling_on_sc=True,
                                             needs_layout_passes=False),
        scratch_types=[
            pltpu.VMEM((SEQ_LEN, bpw), jnp.int32),
            *[pltpu.VMEM((bpw,), jnp.int32) for _ in range(_NSLOT)],
            pltpu.VMEM((bpw,), jnp.float32),
            pltpu.VMEM((bpw,), jnp.float32),
            *[pltpu.SemaphoreType.DMA for _ in range(_NSLOT)],
        ],
    )
    def sc_pool(text_hbm, pk_hbm, out_hbm, idx_v, *rest):
        bufs = rest[:_NSLOT]
        acc0 = rest[_NSLOT]
        acc1 = rest[_NSLOT + 1]
        sems = rest[_NSLOT + 2:]
        wid = lax.axis_index("s") * nc + lax.axis_index("c")
        base = wid * bpw

        pltpu.sync_copy(text_hbm.at[:, pl.ds(base, bpw)], idx_v)

        zero = jnp.zeros((16,), jnp.float32)
        for u in range(bpw // 16):
            acc0[pl.ds(u * 16, 16)] = zero
            acc1[pl.ds(u * 16, 16)] = zero

        hi_mask = jnp.full((16,), -65536, jnp.int32)

        def issue(s, j):
            pltpu.async_copy(pk_hbm.at[idx_v.at[s]], bufs[j], sems[j])

        for j in range(_NSLOT):
            issue(j, j)

        def body(g, carry):
            for j in range(_NSLOT):
                v = g * _NSLOT + j
                pltpu.make_async_copy(
                    pk_hbm.at[idx_v.at[v]], bufs[j], sems[j]).wait()
                for u in range(bpw // 16):
                    sl = pl.ds(u * 16, 16)
                    w = bufs[j][sl]
                    p0 = plsc.bitcast(w << 16, jnp.float32)
                    p1 = plsc.bitcast(w & hi_mask, jnp.float32)
                    plsc.addupdate(acc0.at[sl], p0)
                    plsc.addupdate(acc1.at[sl], p1)

                @pl.when(v + _NSLOT < SEQ_LEN)
                def _():
                    issue(v + _NSLOT, j)

            return carry

        lax.fori_loop(0, SEQ_LEN // _NSLOT, body, 0)

        pltpu.sync_copy(acc0, out_hbm.at[0, pl.ds(base, bpw)])
        pltpu.sync_copy(acc1, out_hbm.at[1, pl.ds(base, bpw)])

    return sc_pool


_sc_pool = None


def kernel(text, embedding, W, b):
    global _sc_pool
    if _sc_pool is None:
        _sc_pool = _make_sc_pool()
    pk = _project(embedding.T, W)
    out2 = _sc_pool(text, pk)
    return out2.T + b
